# baseline (speedup 1.0000x reference)
import functools

import jax
import jax.numpy as jnp
from jax import lax
from jax.experimental import pallas as pl
from jax.experimental.pallas import tpu as pltpu
from jax.experimental.pallas import tpu_sc as plsc

_NC = 2
_NS = 16
_NW = _NC * _NS


def _sc_gather(sid, pid, E_sg, E_pg, E_sm, E_pm):
    B = sid.shape[0]
    D = E_sg.shape[1]
    bpw = B // _NW
    mesh = plsc.VectorSubcoreMesh(core_axis_name="c", subcore_axis_name="s")

    @functools.partial(
        pl.kernel,
        mesh=mesh,
        out_type=[jax.ShapeDtypeStruct((B, D), jnp.float32) for _ in range(4)],
        scratch_types=[
            pltpu.VMEM((bpw,), jnp.int32),
            pltpu.VMEM((bpw,), jnp.int32),
            pltpu.VMEM((bpw, D), jnp.float32),
            pltpu.VMEM((bpw, D), jnp.float32),
            pltpu.VMEM((bpw, D), jnp.float32),
            pltpu.VMEM((bpw, D), jnp.float32),
            pltpu.SemaphoreType.DMA,
        ],
        compiler_params=pltpu.CompilerParams(use_tc_tiling_on_sc=False),
    )
    def gather_kernel(sid_hbm, pid_hbm, esg, epg, esm, epm,
                      o_sg, o_pg, o_sm, o_pm,
                      idx_s, idx_p, r_sg, r_pg, r_sm, r_pm, sem):
        wid = lax.axis_index("s") * _NC + lax.axis_index("c")
        base = wid * bpw
        pltpu.sync_copy(sid_hbm.at[pl.ds(base, bpw)], idx_s)
        pltpu.sync_copy(pid_hbm.at[pl.ds(base, bpw)], idx_p)
        c1 = pltpu.async_copy(esg.at[idx_s], r_sg, sem)
        c2 = pltpu.async_copy(epg.at[idx_p], r_pg, sem)
        c3 = pltpu.async_copy(esm.at[idx_s], r_sm, sem)
        c4 = pltpu.async_copy(epm.at[idx_p], r_pm, sem)
        c1.wait()
        c2.wait()
        c3.wait()
        c4.wait()
        pltpu.sync_copy(r_sg, o_sg.at[pl.ds(base, bpw)])
        pltpu.sync_copy(r_pg, o_pg.at[pl.ds(base, bpw)])
        pltpu.sync_copy(r_sm, o_sm.at[pl.ds(base, bpw)])
        pltpu.sync_copy(r_pm, o_pm.at[pl.ds(base, bpw)])

    return gather_kernel(sid, pid, E_sg, E_pg, E_sm, E_pm)


def _mlp_body(sg, pg, sm, pm, w1, b1, w2, b2, woh, wog, bo, out):
    gmf = sg[:] * pg[:]
    h = jnp.dot(sm[:], w1[:16, :], preferred_element_type=jnp.float32)
    h = h + jnp.dot(pm[:], w1[16:, :], preferred_element_type=jnp.float32)
    h = jnp.maximum(h + b1[:], 0.0)
    h = jnp.maximum(jnp.dot(h, w2[:], preferred_element_type=jnp.float32) + b2[:], 0.0)
    o = jnp.sum(h * woh[:], axis=1) + jnp.sum(gmf * wog[:], axis=1) + bo[0, 0]
    out[:] = jnp.maximum(o, 0.0)


def _tc_mlp(g_sg, g_pg, g_sm, g_pm, W1, b1, W2, b2, Wo, bo):
    B, D = g_sg.shape
    BLK = 2048
    grid = B // BLK
    row_spec = pl.BlockSpec((BLK, D), lambda i: (i, 0))
    rep = lambda shape: pl.BlockSpec(shape, lambda i: (0,) * len(shape))
    woh = Wo[:16, 0].reshape(1, 16)
    wog = Wo[16:, 0].reshape(1, 16)
    return pl.pallas_call(
        _mlp_body,
        grid=(grid,),
        in_specs=[
            row_spec, row_spec, row_spec, row_spec,
            rep((32, 32)), rep((1, 32)), rep((32, 16)), rep((1, 16)),
            rep((1, 16)), rep((1, 16)), rep((1, 1)),
        ],
        out_specs=pl.BlockSpec((BLK,), lambda i: (i,)),
        out_shape=jax.ShapeDtypeStruct((B,), jnp.float32),
    )(g_sg, g_pg, g_sm, g_pm, W1, b1.reshape(1, 32), W2, b2.reshape(1, 16),
      woh, wog, bo.reshape(1, 1))


def kernel(sid, pid, E_sg, E_pg, E_sm, E_pm, W1, b1, W2, b2, Wo, bo):
    g_sg, g_pg, g_sm, g_pm = _sc_gather(sid, pid, E_sg, E_pg, E_sm, E_pm)
    return _tc_mlp(g_sg, g_pg, g_sm, g_pm, W1, b1, W2, b2, Wo, bo)
